# Initial kernel scaffold; baseline (speedup 1.0000x reference)
import jax
import jax.numpy as jnp
from jax.experimental import pallas as pl


def kernel(ed, fa, ad, W_e1, b_e1, W_e2, b_e2, W_f1, b_f1, W_f2, b_f2, W1, b1, g1, be1, W2, b2, g2, be2, W3, b3, g3, be3, W4, b4, g4, be4, Wh, gh, bh):
    raise NotImplementedError("write your pallas kernel here")


# speedup vs baseline: 5.6237x; 5.6237x over previous
import functools

import jax
import jax.numpy as jnp
from jax import lax
from jax.experimental import pallas as pl
from jax.experimental.pallas import tpu as pltpu
from jax.experimental.pallas import tpu_sc as plsc

_B, _F, _NCLS = 2, 10000, 30
_N = _B * _F
_NBLK = 1000
_GRID = _N // _NBLK
_NW = 32
_IDXP = 61440
_BPW = _IDXP // _NW
_CH = 128
_NCH = _BPW // _CH
_EPS = 1e-5


def _gather_rows(table, idx):
  D = table.shape[1]
  mesh = plsc.VectorSubcoreMesh(core_axis_name="c", subcore_axis_name="s")

  @functools.partial(
      pl.kernel,
      out_type=jax.ShapeDtypeStruct((_IDXP, D), jnp.float32),
      mesh=mesh,
      scratch_types=[
          pltpu.VMEM((_BPW,), jnp.int32),
          pltpu.VMEM((_CH, D), jnp.float32),
          pltpu.VMEM((_CH, D), jnp.float32),
          pltpu.SemaphoreType.DMA,
          pltpu.SemaphoreType.DMA,
      ],
  )
  def k(table_hbm, idx_hbm, out_hbm, idx_v, buf0, buf1, sem0, sem1):
    wid = lax.axis_index("s") * 2 + lax.axis_index("c")
    base = wid * _BPW
    pltpu.sync_copy(idx_hbm.at[pl.ds(base, _BPW)], idx_v)
    bufs = (buf0, buf1)
    sems = (sem0, sem1)
    cps = [
        pltpu.async_copy(table_hbm.at[idx_v.at[pl.ds(j * _CH, _CH)]],
                         bufs[j], sems[j])
        for j in range(2)
    ]
    for j in range(_NCH):
      b = j % 2
      cps[b].wait()
      pltpu.sync_copy(bufs[b], out_hbm.at[pl.ds(base + j * _CH, _CH)])
      nj = j + 2
      if nj < _NCH:
        cps[b] = pltpu.async_copy(
            table_hbm.at[idx_v.at[pl.ds(nj * _CH, _CH)]], bufs[b], sems[b])

  return k(table, idx)


def _stem_body(x_ref, we1_ref, be1_ref, we2_ref, be2_ref,
               wf1_ref, bf1_ref, wf2_ref, bf2_ref, out_ref):
  x = x_ref[...]
  e = jnp.maximum(
      jnp.dot(x, we1_ref[...], preferred_element_type=jnp.float32)
      + be1_ref[0:1, :], 0.0)
  e = jnp.maximum(
      jnp.dot(e, we2_ref[...], preferred_element_type=jnp.float32)
      + be2_ref[0:1, :], 0.0)
  f = jnp.maximum(
      jnp.dot(x, wf1_ref[...], preferred_element_type=jnp.float32)
      + bf1_ref[0:1, :], 0.0)
  f = jnp.maximum(
      jnp.dot(f, wf2_ref[...], preferred_element_type=jnp.float32)
      + bf2_ref[0:1, :], 0.0)
  out_ref[...] = jnp.concatenate([e, f], axis=1)


def _stem(x16, we1p, be1, we2t, be2, wf1p, bf1, wf2t, bf2):
  full = lambda r, c: pl.BlockSpec((r, c), lambda i: (0, 0))
  return pl.pallas_call(
      _stem_body,
      grid=(_GRID,),
      in_specs=[
          pl.BlockSpec((_NBLK, 16), lambda i: (i, 0)),
          full(16, 128), full(1, 128), full(128, 64), full(1, 64),
          full(16, 128), full(1, 128), full(128, 64), full(1, 64),
      ],
      out_specs=pl.BlockSpec((_NBLK, 128), lambda i: (i, 0)),
      out_shape=jax.ShapeDtypeStruct((_N, 128), jnp.float32),
  )(x16, we1p, be1.reshape(1, -1), we2t, be2.reshape(1, -1),
    wf1p, bf1.reshape(1, -1), wf2t, bf2.reshape(1, -1))


def _mesh_mm_body(ci, x_ref, n0_ref, n1_ref, n2_ref, a_ref, d_ref,
                  w_ref, b_ref, y_ref, s_ref):
  a = a_ref[0:1, :]
  d = d_ref[0:1, :]
  act = lambda v: jnp.maximum(v * a + d, 0.0)
  y = jnp.dot(act(x_ref[...]), w_ref[0:ci, :],
              preferred_element_type=jnp.float32)
  for k, nref in enumerate((n0_ref, n1_ref, n2_ref)):
    y = y + jnp.dot(act(nref[...]), w_ref[(k + 1) * ci:(k + 2) * ci, :],
                    preferred_element_type=jnp.float32)
  y = y + b_ref[0:1, :]
  y_ref[...] = y

  @pl.when(pl.program_id(0) == 0)
  def _():
    s_ref[...] = jnp.zeros_like(s_ref)

  s_ref[0:1, :] += jnp.sum(y, axis=0, keepdims=True)
  s_ref[1:2, :] += jnp.sum(y * y, axis=0, keepdims=True)


def _mesh_mm(x, nb, a, d, wcat, b):
  ci = x.shape[1]
  co = wcat.shape[1]
  nb_spec = lambda k: pl.BlockSpec((_NBLK, ci), lambda i, k=k: (k * _GRID + i, 0))
  full = lambda r, c: pl.BlockSpec((r, c), lambda i: (0, 0))
  return pl.pallas_call(
      functools.partial(_mesh_mm_body, ci),
      grid=(_GRID,),
      in_specs=[
          pl.BlockSpec((_NBLK, ci), lambda i: (i, 0)),
          nb_spec(0), nb_spec(1), nb_spec(2),
          full(1, ci), full(1, ci), full(4 * ci, co), full(1, co),
      ],
      out_specs=[
          pl.BlockSpec((_NBLK, co), lambda i: (i, 0)),
          pl.BlockSpec((8, co), lambda i: (0, 0)),
      ],
      out_shape=[
          jax.ShapeDtypeStruct((_N, co), jnp.float32),
          jax.ShapeDtypeStruct((8, co), jnp.float32),
      ],
  )(x, nb, nb, nb, a.reshape(1, -1), d.reshape(1, -1), wcat,
    b.reshape(1, -1))


def _head_body(y_ref, a_ref, d_ref, w_ref, s_ref):
  a = a_ref[0:1, :]
  d = d_ref[0:1, :]
  z = jnp.dot(jnp.maximum(y_ref[...] * a + d, 0.0), w_ref[...],
              preferred_element_type=jnp.float32)
  i = pl.program_id(0)

  @pl.when(i == 0)
  def _():
    s_ref[...] = jnp.zeros_like(s_ref)

  zs = jnp.sum(z, axis=0, keepdims=True)

  @pl.when(i < _GRID // 2)
  def _():
    s_ref[0:1, :] += zs

  @pl.when(i >= _GRID // 2)
  def _():
    s_ref[1:2, :] += zs

  s_ref[2:3, :] += jnp.sum(z * z, axis=0, keepdims=True)


def _head(y4, a, d, whp):
  full = lambda r, c: pl.BlockSpec((r, c), lambda i: (0, 0))
  return pl.pallas_call(
      _head_body,
      grid=(_GRID,),
      in_specs=[
          pl.BlockSpec((_NBLK, 512), lambda i: (i, 0)),
          full(1, 512), full(1, 512), full(512, 128),
      ],
      out_specs=pl.BlockSpec((8, 128), lambda i: (0, 0)),
      out_shape=jax.ShapeDtypeStruct((8, 128), jnp.float32),
  )(y4, a.reshape(1, -1), d.reshape(1, -1), whp)


def kernel(ed, fa, ad, W_e1, b_e1, W_e2, b_e2, W_f1, b_f1, W_f2, b_f2,
           W1, b1, g1, be1, W2, b2, g2, be2, W3, b3, g3, be3,
           W4, b4, g4, be4, Wh, gh, bh):
  ed_t = ed.transpose(0, 2, 1).reshape(_N, 3)
  fa_t = fa.transpose(0, 2, 1).reshape(_N, 6)
  x16 = jnp.concatenate(
      [ed_t, fa_t, jnp.zeros((_N, 7), jnp.float32)], axis=1)
  we1p = jnp.zeros((16, 128), jnp.float32).at[:3].set(W_e1.T)
  wf1p = jnp.zeros((16, 128), jnp.float32).at[3:9].set(W_f1.T)

  offs = (jnp.arange(_B, dtype=jnp.int32) * _F)[:, None, None]
  gidx = ad.astype(jnp.int32) + offs
  gidx = gidx.transpose(2, 0, 1).reshape(3 * _N)
  gidx = jnp.concatenate(
      [gidx, jnp.zeros((_IDXP - 3 * _N,), jnp.int32)])

  y = _stem(x16, we1p, b_e1, W_e2.T, b_e2, wf1p, b_f1, W_f2.T, b_f2)
  a = jnp.ones((128,), jnp.float32)
  d = jnp.zeros((128,), jnp.float32)

  for (W, b, g, be) in ((W1, b1, g1, be1), (W2, b2, g2, be2),
                        (W3, b3, g3, be3), (W4, b4, g4, be4)):
    co, ci, _ = W.shape
    wcat = W.transpose(2, 1, 0).reshape(4 * ci, co)
    nb = _gather_rows(y, gidx)
    y, stats = _mesh_mm(y, nb, a, d, wcat, b)
    m = stats[0] / _N
    v = stats[1] / _N - m * m
    r = lax.rsqrt(v + _EPS)
    a = g * r
    d = be - m * g * r

  whp = jnp.zeros((512, 128), jnp.float32).at[:, :_NCLS].set(Wh.T)
  s = _head(y, a, d, whp)
  S0, S1, SS = s[0, :_NCLS], s[1, :_NCLS], s[2, :_NCLS]
  m = (S0 + S1) / (2 * _F)
  v = SS / (2 * _F) - m * m
  r = lax.rsqrt(v + _EPS)
  zm = jnp.stack([S0 / _F, S1 / _F])
  return gh[None] * (zm - m[None]) * r[None] + bh[None]


# speedup vs baseline: 6.1433x; 1.0924x over previous
import functools

import jax
import jax.numpy as jnp
from jax import lax
from jax.experimental import pallas as pl
from jax.experimental.pallas import tpu as pltpu
from jax.experimental.pallas import tpu_sc as plsc

_B, _F, _NCLS = 2, 10000, 30
_N = _B * _F
_NBLK = 1000
_GRID = _N // _NBLK
_NW = 32
_IDXP = 61440
_BPW = _IDXP // _NW
_CH = 128
_NCH = _BPW // _CH
_EPS = 1e-5


def _pack_halves(y):
  c2 = y.shape[1] // 2
  bits = jax.lax.bitcast_convert_type(y, jnp.int32) + 0x8000
  lo = jax.lax.shift_right_logical(bits[:, :c2], 16)
  hi = bits[:, c2:] & jnp.int32(-65536)
  return lo | hi


def _unpack_halves(p):
  lo = jax.lax.bitcast_convert_type(jax.lax.shift_left(p, 16), jnp.float32)
  hi = jax.lax.bitcast_convert_type(p & jnp.int32(-65536), jnp.float32)
  return lo, hi


def _gather_rows(table, idx):
  D = table.shape[1]
  mesh = plsc.VectorSubcoreMesh(core_axis_name="c", subcore_axis_name="s")

  @functools.partial(
      pl.kernel,
      out_type=jax.ShapeDtypeStruct((_IDXP, D), table.dtype),
      mesh=mesh,
      scratch_types=[
          pltpu.VMEM((_BPW,), jnp.int32),
          pltpu.VMEM((_CH, D), table.dtype),
          pltpu.VMEM((_CH, D), table.dtype),
          pltpu.SemaphoreType.DMA,
          pltpu.SemaphoreType.DMA,
      ],
  )
  def k(table_hbm, idx_hbm, out_hbm, idx_v, buf0, buf1, sem0, sem1):
    wid = lax.axis_index("s") * 2 + lax.axis_index("c")
    base = wid * _BPW
    pltpu.sync_copy(idx_hbm.at[pl.ds(base, _BPW)], idx_v)
    bufs = (buf0, buf1)
    sems = (sem0, sem1)
    cps = [
        pltpu.async_copy(table_hbm.at[idx_v.at[pl.ds(j * _CH, _CH)]],
                         bufs[j], sems[j])
        for j in range(2)
    ]
    for j in range(_NCH):
      b = j % 2
      cps[b].wait()
      pltpu.sync_copy(bufs[b], out_hbm.at[pl.ds(base + j * _CH, _CH)])
      nj = j + 2
      if nj < _NCH:
        cps[b] = pltpu.async_copy(
            table_hbm.at[idx_v.at[pl.ds(nj * _CH, _CH)]], bufs[b], sems[b])

  return k(table, idx)


def _stem_body(x_ref, we1_ref, be1_ref, we2_ref, be2_ref,
               wf1_ref, bf1_ref, wf2_ref, bf2_ref, out_ref):
  x = x_ref[...]
  e = jnp.maximum(
      jnp.dot(x, we1_ref[...], preferred_element_type=jnp.float32)
      + be1_ref[0:1, :], 0.0)
  e = jnp.maximum(
      jnp.dot(e, we2_ref[...], preferred_element_type=jnp.float32)
      + be2_ref[0:1, :], 0.0)
  f = jnp.maximum(
      jnp.dot(x, wf1_ref[...], preferred_element_type=jnp.float32)
      + bf1_ref[0:1, :], 0.0)
  f = jnp.maximum(
      jnp.dot(f, wf2_ref[...], preferred_element_type=jnp.float32)
      + bf2_ref[0:1, :], 0.0)
  out_ref[...] = jnp.concatenate([e, f], axis=1)


def _stem(x16, we1p, be1, we2t, be2, wf1p, bf1, wf2t, bf2):
  full = lambda r, c: pl.BlockSpec((r, c), lambda i: (0, 0))
  return pl.pallas_call(
      _stem_body,
      grid=(_GRID,),
      in_specs=[
          pl.BlockSpec((_NBLK, 16), lambda i: (i, 0)),
          full(16, 128), full(1, 128), full(128, 64), full(1, 64),
          full(16, 128), full(1, 128), full(128, 64), full(1, 64),
      ],
      out_specs=pl.BlockSpec((_NBLK, 128), lambda i: (i, 0)),
      out_shape=jax.ShapeDtypeStruct((_N, 128), jnp.float32),
  )(x16, we1p, be1.reshape(1, -1), we2t, be2.reshape(1, -1),
    wf1p, bf1.reshape(1, -1), wf2t, bf2.reshape(1, -1))


def _mesh_mm_body(ci, in_packed, out_packed, x_ref, n0_ref, n1_ref, n2_ref,
                  a_ref, d_ref, w_ref, b_ref, y_ref, s_ref):
  a = a_ref[0:1, :]
  d = d_ref[0:1, :]
  y = b_ref[0:1, :] * jnp.ones((x_ref.shape[0], 1), jnp.float32)
  for k, ref in enumerate((x_ref, n0_ref, n1_ref, n2_ref)):
    if in_packed:
      lo, hi = _unpack_halves(ref[...])
      c2 = ci // 2
      alo = jnp.maximum(lo * a[:, :c2] + d[:, :c2], 0.0)
      ahi = jnp.maximum(hi * a[:, c2:] + d[:, c2:], 0.0)
      y = y + jnp.dot(alo, w_ref[k * ci:k * ci + c2, :],
                      preferred_element_type=jnp.float32)
      y = y + jnp.dot(ahi, w_ref[k * ci + c2:(k + 1) * ci, :],
                      preferred_element_type=jnp.float32)
    else:
      av = jnp.maximum(ref[...] * a + d, 0.0)
      y = y + jnp.dot(av, w_ref[k * ci:(k + 1) * ci, :],
                      preferred_element_type=jnp.float32)
  y_ref[...] = _pack_halves(y) if out_packed else y

  @pl.when(pl.program_id(0) == 0)
  def _():
    s_ref[...] = jnp.zeros_like(s_ref)

  s_ref[0:1, :] += jnp.sum(y, axis=0, keepdims=True)
  s_ref[1:2, :] += jnp.sum(y * y, axis=0, keepdims=True)


def _mesh_mm(x, nb, a, d, wcat, b, in_packed, out_packed):
  ci = wcat.shape[0] // 4
  co = wcat.shape[1]
  cin = ci // 2 if in_packed else ci
  cos = co // 2 if out_packed else co
  odt = jnp.int32 if out_packed else jnp.float32
  nb_spec = lambda k: pl.BlockSpec((_NBLK, cin), lambda i, k=k: (k * _GRID + i, 0))
  full = lambda r, c: pl.BlockSpec((r, c), lambda i: (0, 0))
  return pl.pallas_call(
      functools.partial(_mesh_mm_body, ci, in_packed, out_packed),
      grid=(_GRID,),
      in_specs=[
          pl.BlockSpec((_NBLK, cin), lambda i: (i, 0)),
          nb_spec(0), nb_spec(1), nb_spec(2),
          full(1, ci), full(1, ci), full(4 * ci, co), full(1, co),
      ],
      out_specs=[
          pl.BlockSpec((_NBLK, cos), lambda i: (i, 0)),
          pl.BlockSpec((8, co), lambda i: (0, 0)),
      ],
      out_shape=[
          jax.ShapeDtypeStruct((_N, cos), odt),
          jax.ShapeDtypeStruct((8, co), jnp.float32),
      ],
  )(x, nb, nb, nb, a.reshape(1, -1), d.reshape(1, -1), wcat,
    b.reshape(1, -1))


def _head_body(y_ref, a_ref, d_ref, w_ref, s_ref):
  a = a_ref[0:1, :]
  d = d_ref[0:1, :]
  lo, hi = _unpack_halves(y_ref[...])
  alo = jnp.maximum(lo * a[:, :256] + d[:, :256], 0.0)
  ahi = jnp.maximum(hi * a[:, 256:] + d[:, 256:], 0.0)
  z = (jnp.dot(alo, w_ref[0:256, :], preferred_element_type=jnp.float32)
       + jnp.dot(ahi, w_ref[256:512, :], preferred_element_type=jnp.float32))
  i = pl.program_id(0)

  @pl.when(i == 0)
  def _():
    s_ref[...] = jnp.zeros_like(s_ref)

  zs = jnp.sum(z, axis=0, keepdims=True)

  @pl.when(i < _GRID // 2)
  def _():
    s_ref[0:1, :] += zs

  @pl.when(i >= _GRID // 2)
  def _():
    s_ref[1:2, :] += zs

  s_ref[2:3, :] += jnp.sum(z * z, axis=0, keepdims=True)


def _head(y4p, a, d, whp):
  full = lambda r, c: pl.BlockSpec((r, c), lambda i: (0, 0))
  return pl.pallas_call(
      _head_body,
      grid=(_GRID,),
      in_specs=[
          pl.BlockSpec((_NBLK, 256), lambda i: (i, 0)),
          full(1, 512), full(1, 512), full(512, 128),
      ],
      out_specs=pl.BlockSpec((8, 128), lambda i: (0, 0)),
      out_shape=jax.ShapeDtypeStruct((8, 128), jnp.float32),
  )(y4p, a.reshape(1, -1), d.reshape(1, -1), whp)


def kernel(ed, fa, ad, W_e1, b_e1, W_e2, b_e2, W_f1, b_f1, W_f2, b_f2,
           W1, b1, g1, be1, W2, b2, g2, be2, W3, b3, g3, be3,
           W4, b4, g4, be4, Wh, gh, bh):
  ed_t = ed.transpose(0, 2, 1).reshape(_N, 3)
  fa_t = fa.transpose(0, 2, 1).reshape(_N, 6)
  x16 = jnp.concatenate(
      [ed_t, fa_t, jnp.zeros((_N, 7), jnp.float32)], axis=1)
  we1p = jnp.zeros((16, 128), jnp.float32).at[:3].set(W_e1.T)
  wf1p = jnp.zeros((16, 128), jnp.float32).at[3:9].set(W_f1.T)

  offs = (jnp.arange(_B, dtype=jnp.int32) * _F)[:, None, None]
  gidx = ad.astype(jnp.int32) + offs
  gidx = gidx.transpose(2, 0, 1).reshape(3 * _N)
  gidx = jnp.concatenate(
      [gidx, jnp.zeros((_IDXP - 3 * _N,), jnp.int32)])

  y = _stem(x16, we1p, b_e1, W_e2.T, b_e2, wf1p, b_f1, W_f2.T, b_f2)
  a = jnp.ones((128,), jnp.float32)
  d = jnp.zeros((128,), jnp.float32)

  layers = ((W1, b1, g1, be1, False, False),
            (W2, b2, g2, be2, False, True),
            (W3, b3, g3, be3, True, True),
            (W4, b4, g4, be4, True, True))
  for (W, b, g, be, inp, outp) in layers:
    co, ci, _ = W.shape
    wcat = W.transpose(2, 1, 0).reshape(4 * ci, co)
    nb = _gather_rows(y, gidx)
    y, stats = _mesh_mm(y, nb, a, d, wcat, b, inp, outp)
    m = stats[0] / _N
    v = stats[1] / _N - m * m
    r = lax.rsqrt(v + _EPS)
    a = g * r
    d = be - m * g * r

  whp = jnp.zeros((512, 128), jnp.float32).at[:, :_NCLS].set(Wh.T)
  s = _head(y, a, d, whp)
  S0, S1, SS = s[0, :_NCLS], s[1, :_NCLS], s[2, :_NCLS]
  m = (S0 + S1) / (2 * _F)
  v = SS / (2 * _F) - m * m
  r = lax.rsqrt(v + _EPS)
  zm = jnp.stack([S0 / _F, S1 / _F])
  return gh[None] * (zm - m[None]) * r[None] + bh[None]


# speedup vs baseline: 6.1773x; 1.0055x over previous
import functools

import jax
import jax.numpy as jnp
from jax import lax
from jax.experimental import pallas as pl
from jax.experimental.pallas import tpu as pltpu
from jax.experimental.pallas import tpu_sc as plsc

_B, _F, _NCLS = 2, 10000, 30
_N = _B * _F
_NBLK = 1000
_GRID = _N // _NBLK
_NW = 32
_IDXP = 61440
_BPW = _IDXP // _NW
_CH = 128
_NCH = _BPW // _CH
_EPS = 1e-5


def _pack_halves(y):
  c2 = y.shape[1] // 2
  bits = jax.lax.bitcast_convert_type(y, jnp.int32) + 0x8000
  lo = jax.lax.shift_right_logical(bits[:, :c2], 16)
  hi = bits[:, c2:] & jnp.int32(-65536)
  return lo | hi


def _unpack_halves(p):
  lo = jax.lax.bitcast_convert_type(jax.lax.shift_left(p, 16), jnp.float32)
  hi = jax.lax.bitcast_convert_type(p & jnp.int32(-65536), jnp.float32)
  return lo, hi


def _gather_rows(table, idx):
  D = table.shape[1]
  mesh = plsc.VectorSubcoreMesh(core_axis_name="c", subcore_axis_name="s")

  nbuf = 5

  @functools.partial(
      pl.kernel,
      out_type=jax.ShapeDtypeStruct((_IDXP, D), table.dtype),
      mesh=mesh,
      scratch_types=[
          pltpu.VMEM((_BPW,), jnp.int32),
      ] + [pltpu.VMEM((_CH, D), table.dtype) for _ in range(nbuf)]
        + [pltpu.SemaphoreType.DMA for _ in range(nbuf)],
  )
  def k(table_hbm, idx_hbm, out_hbm, idx_v, *bufsems):
    bufs = bufsems[:nbuf]
    sems = bufsems[nbuf:]
    wid = lax.axis_index("s") * 2 + lax.axis_index("c")
    base = wid * _BPW
    pltpu.sync_copy(idx_hbm.at[pl.ds(base, _BPW)], idx_v)
    cps = [
        pltpu.async_copy(table_hbm.at[idx_v.at[pl.ds(j * _CH, _CH)]],
                         bufs[j], sems[j])
        for j in range(nbuf)
    ]
    for j in range(_NCH):
      b = j % nbuf
      cps[b].wait()
      pltpu.sync_copy(bufs[b], out_hbm.at[pl.ds(base + j * _CH, _CH)])
      nj = j + nbuf
      if nj < _NCH:
        cps[b] = pltpu.async_copy(
            table_hbm.at[idx_v.at[pl.ds(nj * _CH, _CH)]], bufs[b], sems[b])

  return k(table, idx)


def _stem_body(x_ref, we1_ref, be1_ref, we2_ref, be2_ref,
               wf1_ref, bf1_ref, wf2_ref, bf2_ref, out_ref):
  x = x_ref[...]
  e = jnp.maximum(
      jnp.dot(x, we1_ref[...], preferred_element_type=jnp.float32)
      + be1_ref[0:1, :], 0.0)
  e = jnp.maximum(
      jnp.dot(e, we2_ref[...], preferred_element_type=jnp.float32)
      + be2_ref[0:1, :], 0.0)
  f = jnp.maximum(
      jnp.dot(x, wf1_ref[...], preferred_element_type=jnp.float32)
      + bf1_ref[0:1, :], 0.0)
  f = jnp.maximum(
      jnp.dot(f, wf2_ref[...], preferred_element_type=jnp.float32)
      + bf2_ref[0:1, :], 0.0)
  out_ref[...] = jnp.concatenate([e, f], axis=1)


def _stem(x16, we1p, be1, we2t, be2, wf1p, bf1, wf2t, bf2):
  full = lambda r, c: pl.BlockSpec((r, c), lambda i: (0, 0))
  return pl.pallas_call(
      _stem_body,
      grid=(_GRID,),
      in_specs=[
          pl.BlockSpec((_NBLK, 16), lambda i: (i, 0)),
          full(16, 128), full(1, 128), full(128, 64), full(1, 64),
          full(16, 128), full(1, 128), full(128, 64), full(1, 64),
      ],
      out_specs=pl.BlockSpec((_NBLK, 128), lambda i: (i, 0)),
      out_shape=jax.ShapeDtypeStruct((_N, 128), jnp.float32),
  )(x16, we1p, be1.reshape(1, -1), we2t, be2.reshape(1, -1),
    wf1p, bf1.reshape(1, -1), wf2t, bf2.reshape(1, -1))


def _mesh_mm_body(ci, in_packed, out_packed, x_ref, n0_ref, n1_ref, n2_ref,
                  a_ref, d_ref, w_ref, b_ref, y_ref, s_ref):
  a = a_ref[0:1, :]
  d = d_ref[0:1, :]
  y = b_ref[0:1, :] * jnp.ones((x_ref.shape[0], 1), jnp.float32)
  for k, ref in enumerate((x_ref, n0_ref, n1_ref, n2_ref)):
    if in_packed:
      lo, hi = _unpack_halves(ref[...])
      c2 = ci // 2
      alo = jnp.maximum(lo * a[:, :c2] + d[:, :c2], 0.0)
      ahi = jnp.maximum(hi * a[:, c2:] + d[:, c2:], 0.0)
      y = y + jnp.dot(alo, w_ref[k * ci:k * ci + c2, :],
                      preferred_element_type=jnp.float32)
      y = y + jnp.dot(ahi, w_ref[k * ci + c2:(k + 1) * ci, :],
                      preferred_element_type=jnp.float32)
    else:
      av = jnp.maximum(ref[...] * a + d, 0.0)
      y = y + jnp.dot(av, w_ref[k * ci:(k + 1) * ci, :],
                      preferred_element_type=jnp.float32)
  y_ref[...] = _pack_halves(y) if out_packed else y

  @pl.when(pl.program_id(0) == 0)
  def _():
    s_ref[...] = jnp.zeros_like(s_ref)

  s_ref[0:1, :] += jnp.sum(y, axis=0, keepdims=True)
  s_ref[1:2, :] += jnp.sum(y * y, axis=0, keepdims=True)


def _mesh_mm(x, nb, a, d, wcat, b, in_packed, out_packed):
  ci = wcat.shape[0] // 4
  co = wcat.shape[1]
  cin = ci // 2 if in_packed else ci
  cos = co // 2 if out_packed else co
  odt = jnp.int32 if out_packed else jnp.float32
  nb_spec = lambda k: pl.BlockSpec((_NBLK, cin), lambda i, k=k: (k * _GRID + i, 0))
  full = lambda r, c: pl.BlockSpec((r, c), lambda i: (0, 0))
  return pl.pallas_call(
      functools.partial(_mesh_mm_body, ci, in_packed, out_packed),
      grid=(_GRID,),
      in_specs=[
          pl.BlockSpec((_NBLK, cin), lambda i: (i, 0)),
          nb_spec(0), nb_spec(1), nb_spec(2),
          full(1, ci), full(1, ci), full(4 * ci, co), full(1, co),
      ],
      out_specs=[
          pl.BlockSpec((_NBLK, cos), lambda i: (i, 0)),
          pl.BlockSpec((8, co), lambda i: (0, 0)),
      ],
      out_shape=[
          jax.ShapeDtypeStruct((_N, cos), odt),
          jax.ShapeDtypeStruct((8, co), jnp.float32),
      ],
  )(x, nb, nb, nb, a.reshape(1, -1), d.reshape(1, -1), wcat,
    b.reshape(1, -1))


def _head_body(y_ref, a_ref, d_ref, w_ref, s_ref):
  a = a_ref[0:1, :]
  d = d_ref[0:1, :]
  lo, hi = _unpack_halves(y_ref[...])
  alo = jnp.maximum(lo * a[:, :256] + d[:, :256], 0.0)
  ahi = jnp.maximum(hi * a[:, 256:] + d[:, 256:], 0.0)
  z = (jnp.dot(alo, w_ref[0:256, :], preferred_element_type=jnp.float32)
       + jnp.dot(ahi, w_ref[256:512, :], preferred_element_type=jnp.float32))
  i = pl.program_id(0)

  @pl.when(i == 0)
  def _():
    s_ref[...] = jnp.zeros_like(s_ref)

  zs = jnp.sum(z, axis=0, keepdims=True)

  @pl.when(i < _GRID // 2)
  def _():
    s_ref[0:1, :] += zs

  @pl.when(i >= _GRID // 2)
  def _():
    s_ref[1:2, :] += zs

  s_ref[2:3, :] += jnp.sum(z * z, axis=0, keepdims=True)


def _head(y4p, a, d, whp):
  full = lambda r, c: pl.BlockSpec((r, c), lambda i: (0, 0))
  return pl.pallas_call(
      _head_body,
      grid=(_GRID,),
      in_specs=[
          pl.BlockSpec((_NBLK, 256), lambda i: (i, 0)),
          full(1, 512), full(1, 512), full(512, 128),
      ],
      out_specs=pl.BlockSpec((8, 128), lambda i: (0, 0)),
      out_shape=jax.ShapeDtypeStruct((8, 128), jnp.float32),
  )(y4p, a.reshape(1, -1), d.reshape(1, -1), whp)


def kernel(ed, fa, ad, W_e1, b_e1, W_e2, b_e2, W_f1, b_f1, W_f2, b_f2,
           W1, b1, g1, be1, W2, b2, g2, be2, W3, b3, g3, be3,
           W4, b4, g4, be4, Wh, gh, bh):
  ed_t = ed.transpose(0, 2, 1).reshape(_N, 3)
  fa_t = fa.transpose(0, 2, 1).reshape(_N, 6)
  x16 = jnp.concatenate(
      [ed_t, fa_t, jnp.zeros((_N, 7), jnp.float32)], axis=1)
  we1p = jnp.zeros((16, 128), jnp.float32).at[:3].set(W_e1.T)
  wf1p = jnp.zeros((16, 128), jnp.float32).at[3:9].set(W_f1.T)

  offs = (jnp.arange(_B, dtype=jnp.int32) * _F)[:, None, None]
  gidx = ad.astype(jnp.int32) + offs
  gidx = gidx.transpose(2, 0, 1).reshape(3 * _N)
  gidx = jnp.concatenate(
      [gidx, jnp.zeros((_IDXP - 3 * _N,), jnp.int32)])

  y = _stem(x16, we1p, b_e1, W_e2.T, b_e2, wf1p, b_f1, W_f2.T, b_f2)
  a = jnp.ones((128,), jnp.float32)
  d = jnp.zeros((128,), jnp.float32)

  layers = ((W1, b1, g1, be1, False, False),
            (W2, b2, g2, be2, False, True),
            (W3, b3, g3, be3, True, True),
            (W4, b4, g4, be4, True, True))
  for (W, b, g, be, inp, outp) in layers:
    co, ci, _ = W.shape
    wcat = W.transpose(2, 1, 0).reshape(4 * ci, co)
    nb = _gather_rows(y, gidx)
    y, stats = _mesh_mm(y, nb, a, d, wcat, b, inp, outp)
    m = stats[0] / _N
    v = stats[1] / _N - m * m
    r = lax.rsqrt(v + _EPS)
    a = g * r
    d = be - m * g * r

  whp = jnp.zeros((512, 128), jnp.float32).at[:, :_NCLS].set(Wh.T)
  s = _head(y, a, d, whp)
  S0, S1, SS = s[0, :_NCLS], s[1, :_NCLS], s[2, :_NCLS]
  m = (S0 + S1) / (2 * _F)
  v = SS / (2 * _F) - m * m
  r = lax.rsqrt(v + _EPS)
  zm = jnp.stack([S0 / _F, S1 / _F])
  return gh[None] * (zm - m[None]) * r[None] + bh[None]
